# baseline (speedup 1.0000x reference)
import jax
import jax.numpy as jnp
from jax import lax
from jax.experimental import pallas as pl
from jax.experimental.pallas import tpu as pltpu
from jax.experimental.pallas import tpu_sc as plsc

ROWS = 128
COLS = 32768
K = 64
LANES = 16
NV = COLS // LANES
UNROLL = 8
NWORKERS = 32
ROWS_PER = ROWS // NWORKERS
NSEG = 4
SEG_CAP = 16
CAP = LANES * SEG_CAP * NSEG
NCV = CAP // LANES


def _keys(xb):
    sign = jnp.uint32(0x80000000)
    b = lax.bitcast_convert_type(xb, jnp.uint32)
    neg = b >= sign
    return jnp.where(neg, ~b, b | sign)


def _unkey_f(kv):
    sign = jnp.uint32(0x80000000)
    bits = jnp.where(kv >= sign, kv ^ sign, ~kv)
    return plsc.bitcast(bits, jnp.float32)


def _butterfly(v, op):
    lane = lax.iota(jnp.int32, LANES)
    for s in (1, 2, 4, 8):
        v = op(v, v.at[lane ^ s].get(mode="promise_in_bounds"))
    return v


def _process_row(xb, cv, ci, ov, wait_out=None):
    def a_body(i, carry):
        a0, a1, a2, a3 = carry
        for g in range(UNROLL // 4):
            base = (i * UNROLL + g * 4) * LANES
            v0 = xb[pl.ds(base, LANES)]
            v1 = xb[pl.ds(base + LANES, LANES)]
            v2 = xb[pl.ds(base + 2 * LANES, LANES)]
            v3 = xb[pl.ds(base + 3 * LANES, LANES)]
            t = jnp.maximum(jnp.maximum(v0, v1), jnp.maximum(v2, v3))
            m = jnp.maximum(a0, t); t = jnp.minimum(a0, t); a0 = m
            m = jnp.maximum(a1, t); t = jnp.minimum(a1, t); a1 = m
            m = jnp.maximum(a2, t); t = jnp.minimum(a2, t); a2 = m
            a3 = jnp.maximum(a3, t)
        return (a0, a1, a2, a3)

    ninf = [jnp.full((LANES,), -jnp.inf, jnp.float32) for _ in range(4)]
    a0, _, _, a3 = lax.fori_loop(0, NV // UNROLL, a_body, tuple(ninf))
    t2s = _butterfly(a3, jnp.minimum)
    mxs = _butterfly(a0, jnp.maximum)

    def fill_body(i, _):
        for u in range(UNROLL):
            cv[pl.ds((i * UNROLL + u) * LANES, LANES)] = (
                jnp.full((LANES,), -jnp.inf, jnp.float32))
        return 0

    lax.fori_loop(0, NCV // UNROLL, fill_body, 0)

    lane = lax.iota(jnp.int32, LANES)
    step16 = jnp.full((LANES,), LANES, jnp.int32)
    lane_seg = [lane + jnp.full((LANES,), s * LANES * SEG_CAP, jnp.int32)
                for s in range(NSEG)]
    wrap = jnp.full((LANES,), (SEG_CAP - 1) * LANES, jnp.int32)
    inc16 = jnp.full((LANES,), LANES, jnp.int32)
    zero = jnp.zeros((LANES,), jnp.int32)

    def b_body(i, carry):
        offs = list(carry[:NSEG])
        eidx = carry[NSEG]
        for u in range(UNROLL):
            s = u % NSEG
            sl = pl.ds((i * UNROLL + u) * LANES, LANES)
            x = xb[sl]
            msk = x >= t2s
            idx = offs[s] | lane_seg[s]
            plsc.store_scatter(cv, [idx], x, mask=msk)
            plsc.store_scatter(ci, [idx], eidx, mask=msk)
            offs[s] = (offs[s] + jnp.where(msk, inc16, zero)) & wrap
            eidx = eidx + step16
        return (*offs, eidx)

    lax.fori_loop(0, NV // UNROLL, b_body,
                  (zero, zero, zero, zero, lane))

    kmax = _keys(mxs)
    kones = jnp.full((LANES,), 0xFFFFFFFF, dtype=jnp.uint32)
    hi0 = jnp.where(kmax == kones, kmax, kmax + jnp.uint32(1))
    lo0 = _keys(t2s)
    ksp = jnp.full((LANES,), K, jnp.int32)

    def count_ge(thrf):
        acc = jnp.zeros((LANES,), jnp.int32)
        for i in range(NCV):
            acc = acc + (cv[pl.ds(i * LANES, LANES)] >= thrf).astype(
                jnp.int32)
        return _butterfly(acc, jnp.add)

    def s_body(j, carry):
        lo, hi, cnt_hi = carry
        mid = lo + ((hi - lo) >> jnp.uint32(1))
        cnt = count_ge(_unkey_f(mid))
        ge = cnt >= ksp
        return (jnp.where(ge, mid, lo), jnp.where(ge, hi, mid),
                jnp.where(ge, cnt_hi, cnt))

    lo, _, cnt_hi = lax.fori_loop(
        0, 32, s_body, (lo0, hi0, jnp.zeros((LANES,), jnp.int32)))
    thrf = _unkey_f(lo)
    needed = ksp - cnt_hi

    def count_eq_le(ims):
        acc = jnp.zeros((LANES,), jnp.int32)
        for i in range(NCV):
            sl = pl.ds(i * LANES, LANES)
            hit = jnp.logical_and(cv[sl] == thrf, ci[sl] <= ims)
            acc = acc + hit.astype(jnp.int32)
        return _butterfly(acc, jnp.add)

    def t_body(j, carry):
        lo2, hi2 = carry
        mid2 = (lo2 + hi2) >> jnp.int32(1)
        ge2 = count_eq_le(mid2) >= needed
        return (jnp.where(ge2, lo2, mid2 + jnp.int32(1)),
                jnp.where(ge2, mid2, hi2))

    _, istars = lax.fori_loop(
        0, 15, t_body,
        (jnp.zeros((LANES,), jnp.int32),
         jnp.full((LANES,), COLS - 1, jnp.int32)))

    if wait_out is not None:
        wait_out()

    def mask_body(i, _):
        for u in range(UNROLL):
            sl = pl.ds((i * UNROLL + u) * LANES, LANES)
            x = xb[sl]
            ov[sl] = jnp.where(x >= thrf, x, jnp.float32(0.0))
        return 0

    lax.fori_loop(0, NV // UNROLL, mask_body, 0)

    zf = jnp.zeros((LANES,), jnp.float32)
    for i in range(NCV):
        sl = pl.ds(i * LANES, LANES)
        fix = jnp.logical_and(cv[sl] == thrf, ci[sl] > istars)
        plsc.store_scatter(ov, [ci[sl]], zf, mask=fix)


def _body(x_hbm, out_hbm, x0, x1, ov, cv, ci, sin0, sin1, sout):
    wid = lax.axis_index("s") * 2 + lax.axis_index("c")
    row0 = wid * ROWS_PER
    last = jnp.int32(ROWS - 1)

    pltpu.async_copy(x_hbm.at[row0], x0, sin0)
    pltpu.async_copy(x_hbm.at[row0 + 1], x1, sin1)
    pltpu.async_copy(ov, out_hbm.at[row0], sout)

    def half(ip, xb, sin, row_off):
        row = row0 + 2 * ip + row_off
        pltpu.make_async_copy(x_hbm.at[row], xb, sin).wait()

        def wait_out():
            pltpu.make_async_copy(ov, out_hbm.at[row], sout).wait()

        _process_row(xb, cv, ci, ov, wait_out=wait_out)
        pltpu.async_copy(ov, out_hbm.at[row], sout)
        nxt = jnp.minimum(row + 2, last)
        pltpu.async_copy(x_hbm.at[nxt], xb, sin)

    def pair_body(ip, _):
        half(ip, x0, sin0, 0)
        half(ip, x1, sin1, 1)
        return 0

    lax.fori_loop(0, ROWS_PER // 2, pair_body, 0)
    pltpu.make_async_copy(ov, out_hbm.at[row0], sout).wait()
    pltpu.make_async_copy(x_hbm.at[row0], x0, sin0).wait()
    pltpu.make_async_copy(x_hbm.at[row0 + 1], x1, sin1).wait()


def kernel(x):
    mesh = plsc.VectorSubcoreMesh(core_axis_name="c", subcore_axis_name="s")
    f = pl.kernel(
        _body,
        mesh=mesh,
        out_type=jax.ShapeDtypeStruct((ROWS, COLS), jnp.float32),
        scratch_types=[
            pltpu.VMEM((COLS,), jnp.float32),
            pltpu.VMEM((COLS,), jnp.float32),
            pltpu.VMEM((COLS,), jnp.float32),
            pltpu.VMEM((CAP,), jnp.float32),
            pltpu.VMEM((CAP,), jnp.int32),
            pltpu.SemaphoreType.DMA,
            pltpu.SemaphoreType.DMA,
            pltpu.SemaphoreType.DMA,
        ],
        compiler_params=pltpu.CompilerParams(needs_layout_passes=False),
    )
    return f(x)


# speedup vs baseline: 1.1364x; 1.1364x over previous
import jax
import jax.numpy as jnp
from jax import lax
from jax.experimental import pallas as pl
from jax.experimental.pallas import tpu as pltpu
from jax.experimental.pallas import tpu_sc as plsc

ROWS = 128
COLS = 32768
K = 64
LANES = 16
NV = COLS // LANES
UNROLL = 8
NWORKERS = 32
ROWS_PER = ROWS // NWORKERS
NSEG = 2
SEG_CAP = 32
CAP = LANES * SEG_CAP * NSEG
NCV = CAP // LANES


def _keys(xb):
    sign = jnp.uint32(0x80000000)
    b = lax.bitcast_convert_type(xb, jnp.uint32)
    neg = b >= sign
    return jnp.where(neg, ~b, b | sign)


def _unkey_f(kv):
    sign = jnp.uint32(0x80000000)
    bits = jnp.where(kv >= sign, kv ^ sign, ~kv)
    return plsc.bitcast(bits, jnp.float32)


def _butterfly(v, op):
    lane = lax.iota(jnp.int32, LANES)
    for s in (1, 2, 4, 8):
        v = op(v, v.at[lane ^ s].get(mode="promise_in_bounds"))
    return v


def _process_row(xb, cv, ci, ov, wait_out=None):
    def a_body(i, carry):
        a0, a1, a2, a3 = carry
        for g in range(UNROLL // 4):
            base = (i * UNROLL + g * 4) * LANES
            v0 = xb[pl.ds(base, LANES)]
            v1 = xb[pl.ds(base + LANES, LANES)]
            v2 = xb[pl.ds(base + 2 * LANES, LANES)]
            v3 = xb[pl.ds(base + 3 * LANES, LANES)]
            t = jnp.maximum(jnp.maximum(v0, v1), jnp.maximum(v2, v3))
            m = jnp.maximum(a0, t); t = jnp.minimum(a0, t); a0 = m
            m = jnp.maximum(a1, t); t = jnp.minimum(a1, t); a1 = m
            m = jnp.maximum(a2, t); t = jnp.minimum(a2, t); a2 = m
            a3 = jnp.maximum(a3, t)
        return (a0, a1, a2, a3)

    ninf = [jnp.full((LANES,), -jnp.inf, jnp.float32) for _ in range(4)]
    a0, _, _, a3 = lax.fori_loop(0, NV // UNROLL, a_body, tuple(ninf))
    t2s = _butterfly(a3, jnp.minimum)
    mxs = _butterfly(a0, jnp.maximum)

    def fill_body(i, _):
        for u in range(UNROLL):
            cv[pl.ds((i * UNROLL + u) * LANES, LANES)] = (
                jnp.full((LANES,), -jnp.inf, jnp.float32))
        return 0

    lax.fori_loop(0, NCV // UNROLL, fill_body, 0)

    lane = lax.iota(jnp.int32, LANES)
    step16 = jnp.full((LANES,), LANES, jnp.int32)
    lane_seg = [lane + jnp.full((LANES,), s * LANES * SEG_CAP, jnp.int32)
                for s in range(NSEG)]
    wrap = jnp.full((LANES,), (SEG_CAP - 1) * LANES, jnp.int32)
    inc16 = jnp.full((LANES,), LANES, jnp.int32)
    zero = jnp.zeros((LANES,), jnp.int32)

    def b_body(i, carry):
        offs = list(carry[:NSEG])
        eidx = carry[NSEG]
        for u in range(UNROLL):
            s = u % NSEG
            sl = pl.ds((i * UNROLL + u) * LANES, LANES)
            x = xb[sl]
            msk = x >= t2s
            idx = offs[s] | lane_seg[s]
            plsc.store_scatter(cv, [idx], x, mask=msk)
            plsc.store_scatter(ci, [idx], eidx, mask=msk)
            offs[s] = (offs[s] + jnp.where(msk, inc16, zero)) & wrap
            eidx = eidx + step16
        return (*offs, eidx)

    lax.fori_loop(0, NV // UNROLL, b_body,
                  tuple([zero] * NSEG + [lane]))

    kmax = _keys(mxs)
    kones = jnp.full((LANES,), 0xFFFFFFFF, dtype=jnp.uint32)
    hi0 = jnp.where(kmax == kones, kmax, kmax + jnp.uint32(1))
    lo0 = _keys(t2s)
    ksp = jnp.full((LANES,), K, jnp.int32)

    def count_ge(thrf):
        acc = jnp.zeros((LANES,), jnp.int32)
        for i in range(NCV):
            acc = acc + plsc.all_reduce_population_count(
                cv[pl.ds(i * LANES, LANES)] >= thrf)
        return acc

    def s_body(j, carry):
        lo, hi, cnt_hi = carry
        mid = lo + ((hi - lo) >> jnp.uint32(1))
        cnt = count_ge(_unkey_f(mid))
        ge = cnt >= ksp
        return (jnp.where(ge, mid, lo), jnp.where(ge, hi, mid),
                jnp.where(ge, cnt_hi, cnt))

    lo, _, cnt_hi = lax.fori_loop(
        0, 32, s_body, (lo0, hi0, jnp.zeros((LANES,), jnp.int32)))
    thrf = _unkey_f(lo)
    needed = ksp - cnt_hi

    def count_eq_le(ims):
        acc = jnp.zeros((LANES,), jnp.int32)
        for i in range(NCV):
            sl = pl.ds(i * LANES, LANES)
            hit = jnp.logical_and(cv[sl] == thrf, ci[sl] <= ims)
            acc = acc + plsc.all_reduce_population_count(hit)
        return acc

    def t_body(j, carry):
        lo2, hi2 = carry
        mid2 = (lo2 + hi2) >> jnp.int32(1)
        ge2 = count_eq_le(mid2) >= needed
        return (jnp.where(ge2, lo2, mid2 + jnp.int32(1)),
                jnp.where(ge2, mid2, hi2))

    _, istars = lax.fori_loop(
        0, 15, t_body,
        (jnp.zeros((LANES,), jnp.int32),
         jnp.full((LANES,), COLS - 1, jnp.int32)))

    if wait_out is not None:
        wait_out()

    def mask_body(i, _):
        for u in range(UNROLL):
            sl = pl.ds((i * UNROLL + u) * LANES, LANES)
            x = xb[sl]
            ov[sl] = jnp.where(x >= thrf, x, jnp.float32(0.0))
        return 0

    lax.fori_loop(0, NV // UNROLL, mask_body, 0)

    zf = jnp.zeros((LANES,), jnp.float32)
    for i in range(NCV):
        sl = pl.ds(i * LANES, LANES)
        fix = jnp.logical_and(cv[sl] == thrf, ci[sl] > istars)
        plsc.store_scatter(ov, [ci[sl]], zf, mask=fix)


def _body(x_hbm, out_hbm, x0, x1, ov, cv, ci, sin0, sin1, sout):
    wid = lax.axis_index("s") * 2 + lax.axis_index("c")
    row0 = wid * ROWS_PER
    last = jnp.int32(ROWS - 1)

    pltpu.async_copy(x_hbm.at[row0], x0, sin0)
    pltpu.async_copy(x_hbm.at[row0 + 1], x1, sin1)
    pltpu.async_copy(ov, out_hbm.at[row0], sout)

    def half(ip, xb, sin, row_off):
        row = row0 + 2 * ip + row_off
        pltpu.make_async_copy(x_hbm.at[row], xb, sin).wait()

        def wait_out():
            pltpu.make_async_copy(ov, out_hbm.at[row], sout).wait()

        _process_row(xb, cv, ci, ov, wait_out=wait_out)
        pltpu.async_copy(ov, out_hbm.at[row], sout)
        nxt = jnp.minimum(row + 2, last)
        pltpu.async_copy(x_hbm.at[nxt], xb, sin)

    def pair_body(ip, _):
        half(ip, x0, sin0, 0)
        half(ip, x1, sin1, 1)
        return 0

    lax.fori_loop(0, ROWS_PER // 2, pair_body, 0)
    pltpu.make_async_copy(ov, out_hbm.at[row0], sout).wait()
    pltpu.make_async_copy(x_hbm.at[row0], x0, sin0).wait()
    pltpu.make_async_copy(x_hbm.at[row0 + 1], x1, sin1).wait()


def kernel(x):
    mesh = plsc.VectorSubcoreMesh(core_axis_name="c", subcore_axis_name="s")
    f = pl.kernel(
        _body,
        mesh=mesh,
        out_type=jax.ShapeDtypeStruct((ROWS, COLS), jnp.float32),
        scratch_types=[
            pltpu.VMEM((COLS,), jnp.float32),
            pltpu.VMEM((COLS,), jnp.float32),
            pltpu.VMEM((COLS,), jnp.float32),
            pltpu.VMEM((CAP,), jnp.float32),
            pltpu.VMEM((CAP,), jnp.int32),
            pltpu.SemaphoreType.DMA,
            pltpu.SemaphoreType.DMA,
            pltpu.SemaphoreType.DMA,
        ],
        compiler_params=pltpu.CompilerParams(needs_layout_passes=False),
    )
    return f(x)
